# baseline (device time: 119728 ns/iter reference)
import jax
import jax.numpy as jnp
from jax import lax
from jax.experimental import pallas as pl
from jax.experimental.pallas import tpu as pltpu

N_DEV = 16


def kernel(A, B):
    m, _ = A.shape
    _, n = B.shape

    def body(a_ref, b_ref, out_ref, comm_ref, send_sems, recv_sems):
        my = lax.axis_index("i")
        left = (my - 1) % N_DEV
        right = (my + 1) % N_DEV

        barrier_sem = pltpu.get_barrier_semaphore()
        for nbr in (left, right):
            pl.semaphore_signal(
                barrier_sem, inc=1,
                device_id=(nbr,), device_id_type=pl.DeviceIdType.MESH,
            )
        pl.semaphore_wait(barrier_sem, 2)

        a = a_ref[...].astype(jnp.bfloat16)
        b = b_ref[...].astype(jnp.bfloat16)
        partial = jnp.dot(a, b, preferred_element_type=jnp.float32)
        out_ref[...] = partial
        comm_ref[0, :, :] = partial.astype(jnp.bfloat16)

        for h in range(N_DEV - 1):
            send_slot = h % 2
            recv_slot = (h + 1) % 2
            rdma = pltpu.make_async_remote_copy(
                src_ref=comm_ref.at[send_slot],
                dst_ref=comm_ref.at[recv_slot],
                send_sem=send_sems.at[send_slot],
                recv_sem=recv_sems.at[recv_slot],
                device_id=(right,),
                device_id_type=pl.DeviceIdType.MESH,
            )
            rdma.start()
            rdma.wait()
            out_ref[...] += comm_ref[recv_slot, :, :].astype(jnp.float32)

    return pl.pallas_call(
        body,
        out_shape=jax.ShapeDtypeStruct((m, n), jnp.float32),
        in_specs=[
            pl.BlockSpec(memory_space=pltpu.VMEM),
            pl.BlockSpec(memory_space=pltpu.VMEM),
        ],
        out_specs=pl.BlockSpec(memory_space=pltpu.VMEM),
        scratch_shapes=[
            pltpu.VMEM((2, m, n), jnp.bfloat16),
            pltpu.SemaphoreType.DMA((2,)),
            pltpu.SemaphoreType.DMA((2,)),
        ],
        compiler_params=pltpu.CompilerParams(collective_id=0),
    )(A, B)


# device time: 43472 ns/iter; 2.7541x vs baseline; 2.7541x over previous
import jax
import jax.numpy as jnp
from jax import lax
from jax.experimental import pallas as pl
from jax.experimental.pallas import tpu as pltpu

N_DEV = 16
N_STAGES = 4


def kernel(A, B):
    m, _ = A.shape
    _, n = B.shape

    def body(a_ref, b_ref, out_ref, send_bufs, recv_bufs, send_sems, recv_sems):
        my = lax.axis_index("i")
        z = my // 4
        p = my % 4
        partners = [
            4 * z + (p ^ 1),
            4 * z + (3 - p),
            4 * (z ^ 1) + p,
            4 * (z ^ 2) + p,
        ]

        barrier_sem = pltpu.get_barrier_semaphore()
        for tgt in partners:
            pl.semaphore_signal(
                barrier_sem, inc=1,
                device_id=(tgt,), device_id_type=pl.DeviceIdType.MESH,
            )
        pl.semaphore_wait(barrier_sem, N_STAGES)

        a = a_ref[...].astype(jnp.bfloat16)
        b = b_ref[...].astype(jnp.bfloat16)
        out_ref[...] = jnp.dot(a, b, preferred_element_type=jnp.float32)

        for s, tgt in enumerate(partners):
            send_bufs[s, :, :] = out_ref[...].astype(jnp.bfloat16)
            rdma = pltpu.make_async_remote_copy(
                src_ref=send_bufs.at[s],
                dst_ref=recv_bufs.at[s],
                send_sem=send_sems.at[s],
                recv_sem=recv_sems.at[s],
                device_id=(tgt,),
                device_id_type=pl.DeviceIdType.MESH,
            )
            rdma.start()
            rdma.wait()
            out_ref[...] += recv_bufs[s, :, :].astype(jnp.float32)

    return pl.pallas_call(
        body,
        out_shape=jax.ShapeDtypeStruct((m, n), jnp.float32),
        in_specs=[
            pl.BlockSpec(memory_space=pltpu.VMEM),
            pl.BlockSpec(memory_space=pltpu.VMEM),
        ],
        out_specs=pl.BlockSpec(memory_space=pltpu.VMEM),
        scratch_shapes=[
            pltpu.VMEM((N_STAGES, m, n), jnp.bfloat16),
            pltpu.VMEM((N_STAGES, m, n), jnp.bfloat16),
            pltpu.SemaphoreType.DMA((N_STAGES,)),
            pltpu.SemaphoreType.DMA((N_STAGES,)),
        ],
        compiler_params=pltpu.CompilerParams(collective_id=0),
    )(A, B)


# device time: 22917 ns/iter; 5.2244x vs baseline; 1.8969x over previous
import jax
import jax.numpy as jnp
from jax import lax
from jax.experimental import pallas as pl
from jax.experimental.pallas import tpu as pltpu

N_DEV = 16


def kernel(A, B):
    m, _ = A.shape
    _, n = B.shape
    rows = m // N_DEV

    def body(a_ref, b_ref, out_ref, pbf_ref, rs_recv, res,
             rs_send_sems, rs_recv_sems, ag_send_sems, ag_recv_sems):
        my = lax.axis_index("i")

        barrier_sem = pltpu.get_barrier_semaphore()
        for d in range(1, N_DEV):
            pl.semaphore_signal(
                barrier_sem, inc=1,
                device_id=((my + d) % N_DEV,),
                device_id_type=pl.DeviceIdType.MESH,
            )
        pl.semaphore_wait(barrier_sem, N_DEV - 1)

        a = a_ref[...].astype(jnp.bfloat16)
        b = b_ref[...].astype(jnp.bfloat16)
        partial = jnp.dot(a, b, preferred_element_type=jnp.float32)
        out_ref[...] = partial
        pbf_ref[...] = partial.astype(jnp.bfloat16)

        rs_sends = []
        for d in range(1, N_DEV):
            tgt = (my + d) % N_DEV
            rdma = pltpu.make_async_remote_copy(
                src_ref=pbf_ref.at[pl.ds(tgt * rows, rows)],
                dst_ref=rs_recv.at[my],
                send_sem=rs_send_sems.at[d],
                recv_sem=rs_recv_sems.at[my],
                device_id=(tgt,),
                device_id_type=pl.DeviceIdType.MESH,
            )
            rdma.start()
            rs_sends.append(rdma)

        for d in range(1, N_DEV):
            src = (my + d) % N_DEV
            recv = pltpu.make_async_remote_copy(
                src_ref=pbf_ref.at[pl.ds(0, rows)],
                dst_ref=rs_recv.at[src],
                send_sem=rs_send_sems.at[0],
                recv_sem=rs_recv_sems.at[src],
                device_id=(src,),
                device_id_type=pl.DeviceIdType.MESH,
            )
            recv.wait_recv()
            out_ref[pl.ds(my * rows, rows), :] += rs_recv[src].astype(jnp.float32)

        res[pl.ds(my * rows, rows), :] = (
            out_ref[pl.ds(my * rows, rows), :].astype(jnp.bfloat16)
        )

        ag_sends = []
        for d in range(1, N_DEV):
            tgt = (my + d) % N_DEV
            rdma = pltpu.make_async_remote_copy(
                src_ref=res.at[pl.ds(my * rows, rows)],
                dst_ref=res.at[pl.ds(my * rows, rows)],
                send_sem=ag_send_sems.at[d],
                recv_sem=ag_recv_sems.at[my],
                device_id=(tgt,),
                device_id_type=pl.DeviceIdType.MESH,
            )
            rdma.start()
            ag_sends.append(rdma)

        for d in range(1, N_DEV):
            src = (my + d) % N_DEV
            recv = pltpu.make_async_remote_copy(
                src_ref=res.at[pl.ds(0, rows)],
                dst_ref=res.at[pl.ds(src * rows, rows)],
                send_sem=ag_send_sems.at[0],
                recv_sem=ag_recv_sems.at[src],
                device_id=(src,),
                device_id_type=pl.DeviceIdType.MESH,
            )
            recv.wait_recv()

        for rdma in rs_sends:
            rdma.wait_send()
        for rdma in ag_sends:
            rdma.wait_send()

        out_ref[...] = res[...].astype(jnp.float32)

    return pl.pallas_call(
        body,
        out_shape=jax.ShapeDtypeStruct((m, n), jnp.float32),
        in_specs=[
            pl.BlockSpec(memory_space=pltpu.VMEM),
            pl.BlockSpec(memory_space=pltpu.VMEM),
        ],
        out_specs=pl.BlockSpec(memory_space=pltpu.VMEM),
        scratch_shapes=[
            pltpu.VMEM((m, n), jnp.bfloat16),
            pltpu.VMEM((N_DEV, rows, n), jnp.bfloat16),
            pltpu.VMEM((m, n), jnp.bfloat16),
            pltpu.SemaphoreType.DMA((N_DEV,)),
            pltpu.SemaphoreType.DMA((N_DEV,)),
            pltpu.SemaphoreType.DMA((N_DEV,)),
            pltpu.SemaphoreType.DMA((N_DEV,)),
        ],
        compiler_params=pltpu.CompilerParams(collective_id=0),
    )(A, B)


# device time: 22235 ns/iter; 5.3847x vs baseline; 1.0307x over previous
import jax
import jax.numpy as jnp
from jax import lax
from jax.experimental import pallas as pl
from jax.experimental.pallas import tpu as pltpu

N_DEV = 16


def kernel(A, B):
    m, _ = A.shape
    _, n = B.shape
    rows = m // N_DEV

    def body(a_ref, b_ref, out_ref, pbf_ref, rs_recv, acc_ref,
             rs_send_sems, rs_recv_sems, ag_send_sems, ag_recv_sems):
        my = lax.axis_index("i")

        barrier_sem = pltpu.get_barrier_semaphore()
        for d in range(1, N_DEV):
            pl.semaphore_signal(
                barrier_sem, inc=1,
                device_id=((my + d) % N_DEV,),
                device_id_type=pl.DeviceIdType.MESH,
            )

        a = a_ref[...].astype(jnp.bfloat16)
        b = b_ref[...].astype(jnp.bfloat16)
        pbf_ref[...] = jnp.dot(
            a, b, preferred_element_type=jnp.float32
        ).astype(jnp.bfloat16)

        pl.semaphore_wait(barrier_sem, N_DEV - 1)

        rs_sends = []
        for d in range(1, N_DEV):
            tgt = (my + d) % N_DEV
            rdma = pltpu.make_async_remote_copy(
                src_ref=pbf_ref.at[pl.ds(tgt * rows, rows)],
                dst_ref=rs_recv.at[my],
                send_sem=rs_send_sems.at[d],
                recv_sem=rs_recv_sems.at[my],
                device_id=(tgt,),
                device_id_type=pl.DeviceIdType.MESH,
            )
            rdma.start()
            rs_sends.append(rdma)

        acc_ref[...] = pbf_ref[pl.ds(my * rows, rows), :].astype(jnp.float32)
        for d in range(1, N_DEV):
            src = (my + d) % N_DEV
            recv = pltpu.make_async_remote_copy(
                src_ref=pbf_ref.at[pl.ds(0, rows)],
                dst_ref=rs_recv.at[src],
                send_sem=rs_send_sems.at[0],
                recv_sem=rs_recv_sems.at[src],
                device_id=(src,),
                device_id_type=pl.DeviceIdType.MESH,
            )
            recv.wait_recv()
            acc_ref[...] += rs_recv[src].astype(jnp.float32)

        out_ref[pl.ds(my * rows, rows), :] = acc_ref[...].astype(jnp.bfloat16)

        ag_sends = []
        for d in range(1, N_DEV):
            tgt = (my + d) % N_DEV
            rdma = pltpu.make_async_remote_copy(
                src_ref=out_ref.at[pl.ds(my * rows, rows)],
                dst_ref=out_ref.at[pl.ds(my * rows, rows)],
                send_sem=ag_send_sems.at[d],
                recv_sem=ag_recv_sems.at[my],
                device_id=(tgt,),
                device_id_type=pl.DeviceIdType.MESH,
            )
            rdma.start()
            ag_sends.append(rdma)

        for d in range(1, N_DEV):
            src = (my + d) % N_DEV
            recv = pltpu.make_async_remote_copy(
                src_ref=out_ref.at[pl.ds(0, rows)],
                dst_ref=out_ref.at[pl.ds(src * rows, rows)],
                send_sem=ag_send_sems.at[0],
                recv_sem=ag_recv_sems.at[src],
                device_id=(src,),
                device_id_type=pl.DeviceIdType.MESH,
            )
            recv.wait_recv()

        for rdma in rs_sends:
            rdma.wait_send()
        for rdma in ag_sends:
            rdma.wait_send()

    return pl.pallas_call(
        body,
        out_shape=jax.ShapeDtypeStruct((m, n), jnp.bfloat16),
        in_specs=[
            pl.BlockSpec(memory_space=pltpu.VMEM),
            pl.BlockSpec(memory_space=pltpu.VMEM),
        ],
        out_specs=pl.BlockSpec(memory_space=pltpu.VMEM),
        scratch_shapes=[
            pltpu.VMEM((m, n), jnp.bfloat16),
            pltpu.VMEM((N_DEV, rows, n), jnp.bfloat16),
            pltpu.VMEM((rows, n), jnp.float32),
            pltpu.SemaphoreType.DMA((N_DEV,)),
            pltpu.SemaphoreType.DMA((N_DEV,)),
            pltpu.SemaphoreType.DMA((N_DEV,)),
            pltpu.SemaphoreType.DMA((N_DEV,)),
        ],
        compiler_params=pltpu.CompilerParams(collective_id=0),
    )(A, B)


# device time: 20018 ns/iter; 5.9810x vs baseline; 1.1108x over previous
import jax
import jax.numpy as jnp
from jax import lax
from jax.experimental import pallas as pl
from jax.experimental.pallas import tpu as pltpu

N_DEV = 16


def kernel(A, B):
    m, _ = A.shape
    _, n = B.shape
    rows = m // N_DEV

    def body(a_ref, b_ref, out_ref, pbf_ref, rs_recv, acc_ref,
             rs_send_sems, rs_recv_sems, ag_send_sems, ag_recv_sems):
        my = lax.axis_index("i")

        barrier_sem = pltpu.get_barrier_semaphore()
        for d in range(1, N_DEV):
            pl.semaphore_signal(
                barrier_sem, inc=1,
                device_id=((my + d) % N_DEV,),
                device_id_type=pl.DeviceIdType.MESH,
            )

        pbf_ref[...] = jnp.dot(
            a_ref[...], b_ref[...], preferred_element_type=jnp.float32
        ).astype(jnp.bfloat16)

        pl.semaphore_wait(barrier_sem, N_DEV - 1)

        rs_sends = []
        for d in range(1, N_DEV):
            tgt = (my + d) % N_DEV
            rdma = pltpu.make_async_remote_copy(
                src_ref=pbf_ref.at[pl.ds(tgt * rows, rows)],
                dst_ref=rs_recv.at[my],
                send_sem=rs_send_sems.at[d],
                recv_sem=rs_recv_sems.at[my],
                device_id=(tgt,),
                device_id_type=pl.DeviceIdType.MESH,
            )
            rdma.start()
            rs_sends.append(rdma)

        acc_ref[...] = pbf_ref[pl.ds(my * rows, rows), :].astype(jnp.float32)
        for d in range(1, N_DEV):
            src = (my + d) % N_DEV
            recv = pltpu.make_async_remote_copy(
                src_ref=pbf_ref.at[pl.ds(0, rows)],
                dst_ref=rs_recv.at[src],
                send_sem=rs_send_sems.at[0],
                recv_sem=rs_recv_sems.at[src],
                device_id=(src,),
                device_id_type=pl.DeviceIdType.MESH,
            )
            recv.wait_recv()
            acc_ref[...] += rs_recv[src].astype(jnp.float32)

        out_ref[pl.ds(my * rows, rows), :] = acc_ref[...].astype(jnp.bfloat16)

        ag_sends = []
        for d in range(1, N_DEV):
            tgt = (my + d) % N_DEV
            rdma = pltpu.make_async_remote_copy(
                src_ref=out_ref.at[pl.ds(my * rows, rows)],
                dst_ref=out_ref.at[pl.ds(my * rows, rows)],
                send_sem=ag_send_sems.at[d],
                recv_sem=ag_recv_sems.at[my],
                device_id=(tgt,),
                device_id_type=pl.DeviceIdType.MESH,
            )
            rdma.start()
            ag_sends.append(rdma)

        for d in range(1, N_DEV):
            src = (my + d) % N_DEV
            recv = pltpu.make_async_remote_copy(
                src_ref=out_ref.at[pl.ds(0, rows)],
                dst_ref=out_ref.at[pl.ds(src * rows, rows)],
                send_sem=ag_send_sems.at[0],
                recv_sem=ag_recv_sems.at[src],
                device_id=(src,),
                device_id_type=pl.DeviceIdType.MESH,
            )
            recv.wait_recv()

        for rdma in rs_sends:
            rdma.wait_send()
        for rdma in ag_sends:
            rdma.wait_send()

    return pl.pallas_call(
        body,
        out_shape=jax.ShapeDtypeStruct((m, n), jnp.bfloat16),
        in_specs=[
            pl.BlockSpec(memory_space=pltpu.VMEM),
            pl.BlockSpec(memory_space=pltpu.VMEM),
        ],
        out_specs=pl.BlockSpec(memory_space=pltpu.VMEM),
        scratch_shapes=[
            pltpu.VMEM((m, n), jnp.bfloat16),
            pltpu.VMEM((N_DEV, rows, n), jnp.bfloat16),
            pltpu.VMEM((rows, n), jnp.float32),
            pltpu.SemaphoreType.DMA((N_DEV,)),
            pltpu.SemaphoreType.DMA((N_DEV,)),
            pltpu.SemaphoreType.DMA((N_DEV,)),
            pltpu.SemaphoreType.DMA((N_DEV,)),
        ],
        compiler_params=pltpu.CompilerParams(collective_id=0),
    )(A, B)
